# Initial kernel scaffold; baseline (speedup 1.0000x reference)
import jax
import jax.numpy as jnp
from jax.experimental import pallas as pl


def kernel(x, router_probs, fc1, fc2, fc3):
    raise NotImplementedError("write your pallas kernel here")


# speedup vs baseline: 2.0804x; 2.0804x over previous
import functools

import jax
import jax.numpy as jnp
from jax import lax
from jax.experimental import pallas as pl
from jax.experimental.pallas import tpu as pltpu
from jax.experimental.pallas import tpu_sc as plsc

NUM_EXPERTS = 8
TOPK = 2
HIDDEN = 768
INTER = 3072
NUM_ROWS = 2048

BLK = 128
NB = (NUM_ROWS * TOPK) // BLK + (NUM_EXPERTS - 1)
S = NB * BLK

NW = 32
RPW = NUM_ROWS // NW

_NEG = -3.0e38
_HI = lax.Precision.HIGHEST


def _routing_body(rp_ref, pos0_ref, pos1_ref, p0_ref, p1_ref, be_ref, na_ref):
    i32 = jnp.int32
    r = rp_ref[...]
    iota_e = lax.broadcasted_iota(i32, r.shape, 1)

    m0 = jnp.max(r, axis=1, keepdims=True)
    e0 = jnp.min(jnp.where(r == m0, iota_e, NUM_EXPERTS), axis=1, keepdims=True)
    rm = jnp.where(iota_e == e0, _NEG, r)
    m1 = jnp.max(rm, axis=1, keepdims=True)
    e1 = jnp.min(jnp.where(rm == m1, iota_e, NUM_EXPERTS), axis=1, keepdims=True)

    p0 = 1.0 / (1.0 + jnp.exp(m1 - m0))
    p0_ref[...] = p0
    p1_ref[...] = 1.0 - p0

    oh0 = (iota_e == e0).astype(i32)
    oh1 = (iota_e == e1).astype(i32)

    def inc_cumsum(c):
        k = 1
        while k < NUM_ROWS:
            z = jnp.zeros((k, NUM_EXPERTS), i32)
            c = c + jnp.concatenate(
                [z, lax.slice(c, (0, 0), (NUM_ROWS - k, NUM_EXPERTS))], axis=0)
            k *= 2
        return c

    cum0 = inc_cumsum(oh0)
    cum1 = inc_cumsum(oh1)
    tot0 = lax.slice(cum0, (NUM_ROWS - 1, 0), (NUM_ROWS, NUM_EXPERTS))
    tot1 = lax.slice(cum1, (NUM_ROWS - 1, 0), (NUM_ROWS, NUM_EXPERTS))
    counts = tot0 + tot1
    padded = ((counts + (BLK - 1)) // BLK) * BLK

    def ex_cumsum_lanes(c):
        k = 1
        while k < NUM_EXPERTS:
            z = jnp.zeros((1, k), i32)
            c = c + jnp.concatenate(
                [z, lax.slice(c, (0, 0), (1, NUM_EXPERTS - k))], axis=1)
            k *= 2
        z = jnp.zeros((1, 1), i32)
        return jnp.concatenate(
            [z, lax.slice(c, (0, 0), (1, NUM_EXPERTS - 1))], axis=1)

    start = ex_cumsum_lanes(padded)
    nxt = start + padded

    start0 = jnp.sum(oh0 * start, axis=1, keepdims=True)
    start1 = jnp.sum(oh1 * (start + tot0), axis=1, keepdims=True)
    rank0 = jnp.sum(oh0 * (cum0 - oh0), axis=1, keepdims=True)
    rank1 = jnp.sum(oh1 * (cum1 - oh1), axis=1, keepdims=True)
    pos0_ref[...] = start0 + rank0
    pos1_ref[...] = start1 + rank1

    bi = lax.broadcasted_iota(i32, (NB, NUM_EXPERTS), 0) * BLK
    cmp = (bi >= nxt).astype(i32)
    be = jnp.sum(cmp, axis=1, keepdims=True)
    be_ref[...] = jnp.minimum(be, NUM_EXPERTS - 1)
    na_ref[...] = jnp.sum(padded, axis=1, keepdims=True) // BLK


@functools.lru_cache(maxsize=None)
def _sc_kernels():
    mesh = plsc.VectorSubcoreMesh(core_axis_name="c", subcore_axis_name="s")

    @functools.partial(
        pl.kernel,
        out_type=jax.ShapeDtypeStruct((S, HIDDEN), jnp.float32),
        mesh=mesh,
        scratch_types=[
            pltpu.VMEM((RPW,), jnp.int32),
            pltpu.VMEM((RPW,), jnp.int32),
            pltpu.VMEM((RPW, HIDDEN), jnp.float32),
            pltpu.SemaphoreType.DMA,
        ],
    )
    def dispatch(x_hbm, pos0_hbm, pos1_hbm, xs_hbm, i0_v, i1_v, rows_v, sem):
        wid = lax.axis_index("s") * 2 + lax.axis_index("c")
        base = wid * RPW
        pltpu.sync_copy(pos0_hbm.at[pl.ds(base, RPW)], i0_v)
        pltpu.sync_copy(pos1_hbm.at[pl.ds(base, RPW)], i1_v)
        pltpu.sync_copy(x_hbm.at[pl.ds(base, RPW)], rows_v)
        c0 = pltpu.async_copy(rows_v, xs_hbm.at[i0_v], sem)
        c1 = pltpu.async_copy(rows_v, xs_hbm.at[i1_v], sem)
        c0.wait(); c1.wait()

    @functools.partial(
        pl.kernel,
        out_type=[
            jax.ShapeDtypeStruct((NUM_ROWS, HIDDEN), jnp.float32),
            jax.ShapeDtypeStruct((NUM_ROWS, HIDDEN), jnp.float32),
        ],
        mesh=mesh,
        scratch_types=[
            pltpu.VMEM((RPW,), jnp.int32),
            pltpu.VMEM((RPW,), jnp.int32),
            pltpu.VMEM((RPW, HIDDEN), jnp.float32),
            pltpu.VMEM((RPW, HIDDEN), jnp.float32),
            pltpu.SemaphoreType.DMA,
        ],
    )
    def combine(ysw_hbm, pos0_hbm, pos1_hbm, y0_hbm, y1_hbm, i0_v, i1_v,
                b0_v, b1_v, sem):
        wid = lax.axis_index("s") * 2 + lax.axis_index("c")
        base = wid * RPW
        pltpu.sync_copy(pos0_hbm.at[pl.ds(base, RPW)], i0_v)
        pltpu.sync_copy(pos1_hbm.at[pl.ds(base, RPW)], i1_v)
        g0 = pltpu.async_copy(ysw_hbm.at[i0_v], b0_v, sem)
        g1 = pltpu.async_copy(ysw_hbm.at[i1_v], b1_v, sem)
        g0.wait(); g1.wait()
        pltpu.sync_copy(b0_v, y0_hbm.at[pl.ds(base, RPW)])
        pltpu.sync_copy(b1_v, y1_hbm.at[pl.ds(base, RPW)])

    return dispatch, combine


def _ffn_body(be_ref, na_ref, xs_ref, fc1_ref, fc3_ref, fc2_ref, ys_ref):
    b = pl.program_id(0)

    @pl.when(b < na_ref[0])
    def _do():
        xs = xs_ref[...]
        h1 = jnp.dot(xs, fc1_ref[0], preferred_element_type=jnp.float32)
        h3 = jnp.dot(xs, fc3_ref[0], preferred_element_type=jnp.float32)
        act = (h1 / (1.0 + jnp.exp(-h1))) * h3
        ys_ref[...] = jnp.dot(act, fc2_ref[0],
                              preferred_element_type=jnp.float32)


def _mix_body(p0_ref, p1_ref, y0_ref, y1_ref, out_ref):
    out_ref[...] = p0_ref[...] * y0_ref[...] + p1_ref[...] * y1_ref[...]


@jax.jit
def kernel(x, router_probs, fc1, fc2, fc3):
    f32 = jnp.float32
    i32 = jnp.int32
    pos0, pos1, p0, p1, be, na = pl.pallas_call(
        _routing_body,
        out_shape=[
            jax.ShapeDtypeStruct((NUM_ROWS, 1), i32),
            jax.ShapeDtypeStruct((NUM_ROWS, 1), i32),
            jax.ShapeDtypeStruct((NUM_ROWS, 1), f32),
            jax.ShapeDtypeStruct((NUM_ROWS, 1), f32),
            jax.ShapeDtypeStruct((NB, 1), i32),
            jax.ShapeDtypeStruct((1, 1), i32),
        ],
    )(router_probs)

    pos0_f = jnp.reshape(pos0, (NUM_ROWS,))
    pos1_f = jnp.reshape(pos1, (NUM_ROWS,))
    dispatch, combine = _sc_kernels()
    xs = dispatch(x, pos0_f, pos1_f)

    be_flat = jnp.reshape(be, (NB,))
    na_flat = jnp.reshape(na, (1,))
    grid_spec = pltpu.PrefetchScalarGridSpec(
        num_scalar_prefetch=2,
        grid=(NB,),
        in_specs=[
            pl.BlockSpec((BLK, HIDDEN), lambda b, be, na: (b, 0)),
            pl.BlockSpec((1, HIDDEN, INTER),
                         lambda b, be, na: (be[jnp.minimum(b, na[0] - 1)],
                                            0, 0)),
            pl.BlockSpec((1, HIDDEN, INTER),
                         lambda b, be, na: (be[jnp.minimum(b, na[0] - 1)],
                                            0, 0)),
            pl.BlockSpec((1, INTER, HIDDEN),
                         lambda b, be, na: (be[jnp.minimum(b, na[0] - 1)],
                                            0, 0)),
        ],
        out_specs=pl.BlockSpec((BLK, HIDDEN), lambda b, be, na: (b, 0)),
    )
    ysw = pl.pallas_call(
        _ffn_body,
        grid_spec=grid_spec,
        out_shape=jax.ShapeDtypeStruct((S, HIDDEN), f32),
        compiler_params=pltpu.CompilerParams(
            dimension_semantics=("arbitrary",),
            vmem_limit_bytes=100 * 1024 * 1024),
    )(be_flat, na_flat, xs, fc1, fc3, fc2)

    y0, y1 = combine(ysw, pos0_f, pos1_f)

    out = pl.pallas_call(
        _mix_body,
        out_shape=jax.ShapeDtypeStruct((NUM_ROWS, HIDDEN), f32),
    )(p0, p1, y0, y1)
    return out
